# baseline (speedup 1.0000x reference)
import functools

import jax
import jax.numpy as jnp
from jax.experimental import pallas as pl
from jax.experimental.pallas import tpu as pltpu

_NBUF = 3
_CH = 80


def _chunk_copy(a_hbm, bufs, sems, chunk, slot, ch):
    return pltpu.make_async_copy(
        a_hbm.at[pl.ds(chunk * ch, ch), :],
        bufs.at[slot],
        sems.at[slot],
    )


def _body(nch, ch, f_ref, w_ref, b_ref, a_hbm, s_ref, loss_ref,
          bufs, d_acc, t_acc, sems):
    for q in range(min(_NBUF, nch)):
        _chunk_copy(a_hbm, bufs, sems, q, q, ch).start()

    logits = jax.lax.dot_general(
        f_ref[...], w_ref[...],
        dimension_numbers=(((1,), (1,)), ((), ())),
        preferred_element_type=jnp.float32,
    ) + b_ref[...]
    mx = jnp.max(logits, axis=1, keepdims=True)
    e = jnp.exp(logits - mx)
    s_ref[...] = e / jnp.sum(e, axis=1, keepdims=True)
    s = s_ref[...]

    d_acc[...] = jnp.zeros_like(d_acc)
    t_acc[...] = jnp.zeros_like(t_acc)

    def step(c, carry):
        slot = jax.lax.rem(c, _NBUF)
        _chunk_copy(a_hbm, bufs, sems, c, slot, ch).wait()
        a = bufs[slot]
        d_acc[...] += jnp.sum(a, axis=0, keepdims=True)
        m = jnp.dot(a, s, preferred_element_type=jnp.float32)
        s_blk = s_ref[pl.ds(c * ch, ch), :]
        t_acc[...] += jnp.full((1, 1), jnp.sum(m * s_blk), jnp.float32)

        @pl.when(c + _NBUF < nch)
        def _prefetch():
            _chunk_copy(a_hbm, bufs, sems, c + _NBUF, slot, ch).start()

        return carry

    jax.lax.fori_loop(0, nch, step, 0, unroll=False)

    d = d_acc[...]
    edges = jnp.sum(d)
    std = jnp.dot(d, s, preferred_element_type=jnp.float32)
    trace_norm = jnp.sum(std * std) / (2.0 * edges)
    loss = -(t_acc[0, 0] - trace_norm) / (2.0 * edges)
    loss_ref[...] = jnp.full((1, 1), loss, jnp.float32)


@jax.jit
def kernel(features, adjacency, W, b):
    n, d_feat = features.shape
    k = W.shape[0]

    ch = _CH
    if n % ch != 0:
        ch = n
    nch = n // ch

    assignments, loss = pl.pallas_call(
        functools.partial(_body, nch, ch),
        in_specs=[
            pl.BlockSpec(memory_space=pltpu.MemorySpace.VMEM),
            pl.BlockSpec(memory_space=pltpu.MemorySpace.VMEM),
            pl.BlockSpec(memory_space=pltpu.MemorySpace.VMEM),
            pl.BlockSpec(memory_space=pl.ANY),
        ],
        out_specs=[
            pl.BlockSpec(memory_space=pltpu.MemorySpace.VMEM),
            pl.BlockSpec(memory_space=pltpu.MemorySpace.VMEM),
        ],
        out_shape=[
            jax.ShapeDtypeStruct((n, k), jnp.float32),
            jax.ShapeDtypeStruct((1, 1), jnp.float32),
        ],
        scratch_shapes=[
            pltpu.VMEM((_NBUF, ch, n), jnp.float32),
            pltpu.VMEM((1, n), jnp.float32),
            pltpu.VMEM((1, 1), jnp.float32),
            pltpu.SemaphoreType.DMA((_NBUF,)),
        ],
    )(features, W, b.reshape(1, k), adjacency)

    return assignments, loss[0, 0]


# speedup vs baseline: 1.0396x; 1.0396x over previous
import functools

import jax
import jax.numpy as jnp
from jax.experimental import pallas as pl
from jax.experimental.pallas import tpu as pltpu

_NBUF = 3
_CH = 200


def _chunk_copies(a_hbm, bufs, sems, sems2, chunk, slot, ch):
    h = (ch // 16) * 8
    return (
        pltpu.make_async_copy(
            a_hbm.at[pl.ds(chunk * ch, h), :],
            bufs.at[slot, pl.ds(0, h), :],
            sems.at[slot],
        ),
        pltpu.make_async_copy(
            a_hbm.at[pl.ds(chunk * ch + h, ch - h), :],
            bufs.at[slot, pl.ds(h, ch - h), :],
            sems2.at[slot],
        ),
    )


def _body(nch, ch, f_ref, w_ref, b_ref, a_hbm, s_ref, loss_ref,
          bufs, d_acc, t_acc, sems, sems2):
    for q in range(min(_NBUF, nch)):
        for cp in _chunk_copies(a_hbm, bufs, sems, sems2, q, q, ch):
            cp.start()

    logits = jax.lax.dot_general(
        f_ref[...], w_ref[...],
        dimension_numbers=(((1,), (1,)), ((), ())),
        preferred_element_type=jnp.float32,
    ) + b_ref[...]
    mx = jnp.max(logits, axis=1, keepdims=True)
    e = jnp.exp(logits - mx)
    s_ref[...] = e / jnp.sum(e, axis=1, keepdims=True)
    s = s_ref[...]

    d_acc[...] = jnp.zeros_like(d_acc)
    t_acc[...] = jnp.zeros_like(t_acc)

    def step(c, carry):
        slot = jax.lax.rem(c, _NBUF)
        for cp in _chunk_copies(a_hbm, bufs, sems, sems2, c, slot, ch):
            cp.wait()
        a = bufs[slot]
        d_acc[...] += jnp.sum(a, axis=0, keepdims=True)
        m = jnp.dot(a, s, preferred_element_type=jnp.float32)
        s_blk = s_ref[pl.ds(c * ch, ch), :]
        t_acc[...] += jnp.full((1, 1), jnp.sum(m * s_blk), jnp.float32)

        @pl.when(c + _NBUF < nch)
        def _prefetch():
            for cp in _chunk_copies(a_hbm, bufs, sems, sems2, c + _NBUF, slot, ch):
                cp.start()

        return carry

    jax.lax.fori_loop(0, nch, step, 0, unroll=False)

    d = d_acc[...]
    edges = jnp.sum(d)
    std = jnp.dot(d, s, preferred_element_type=jnp.float32)
    trace_norm = jnp.sum(std * std) / (2.0 * edges)
    loss = -(t_acc[0, 0] - trace_norm) / (2.0 * edges)
    loss_ref[...] = jnp.full((1, 1), loss, jnp.float32)


@jax.jit
def kernel(features, adjacency, W, b):
    n, d_feat = features.shape
    k = W.shape[0]

    ch = _CH
    if n % ch != 0:
        ch = n
    nch = n // ch

    assignments, loss = pl.pallas_call(
        functools.partial(_body, nch, ch),
        in_specs=[
            pl.BlockSpec(memory_space=pltpu.MemorySpace.VMEM),
            pl.BlockSpec(memory_space=pltpu.MemorySpace.VMEM),
            pl.BlockSpec(memory_space=pltpu.MemorySpace.VMEM),
            pl.BlockSpec(memory_space=pl.ANY),
        ],
        out_specs=[
            pl.BlockSpec(memory_space=pltpu.MemorySpace.VMEM),
            pl.BlockSpec(memory_space=pltpu.MemorySpace.VMEM),
        ],
        out_shape=[
            jax.ShapeDtypeStruct((n, k), jnp.float32),
            jax.ShapeDtypeStruct((1, 1), jnp.float32),
        ],
        scratch_shapes=[
            pltpu.VMEM((_NBUF, ch, n), jnp.float32),
            pltpu.VMEM((1, n), jnp.float32),
            pltpu.VMEM((1, 1), jnp.float32),
            pltpu.SemaphoreType.DMA((_NBUF,)),
            pltpu.SemaphoreType.DMA((_NBUF,)),
        ],
    )(features, W, b.reshape(1, k), adjacency)

    return assignments, loss[0, 0]
